# Initial kernel scaffold; baseline (speedup 1.0000x reference)
import jax
import jax.numpy as jnp
from jax.experimental import pallas as pl


def kernel(idx, pe):
    raise NotImplementedError("write your pallas kernel here")


# speedup vs baseline: 3.9679x; 3.9679x over previous
import functools

import jax
import jax.numpy as jnp
from jax import lax
from jax.experimental import pallas as pl
from jax.experimental.pallas import tpu as pltpu
from jax.experimental.pallas import tpu_sc as plsc

_EMB_DIM = 64
_NC = 2
_NS = 16
_NW = _NC * _NS


def _make_gather(B: int, D: int, chunk: int):
    assert B % (_NW * chunk) == 0
    b_per_w = B // _NW
    n_chunks = b_per_w // chunk
    mesh = plsc.VectorSubcoreMesh(core_axis_name="c", subcore_axis_name="s")

    @functools.partial(
        pl.kernel,
        mesh=mesh,
        out_type=jax.ShapeDtypeStruct((B, D), jnp.float32),
        scratch_types=[
            pltpu.VMEM((chunk,), jnp.int32),
            pltpu.VMEM((chunk, D), jnp.float32),
            pltpu.SemaphoreType.DMA,
        ],
        compiler_params=pltpu.CompilerParams(use_tc_tiling_on_sc=False),
    )
    def gather_kernel(idx_hbm, table_hbm, out_hbm, idx_v, rows_v, sem):
        wid = lax.axis_index("s") * _NC + lax.axis_index("c")
        base = wid * b_per_w

        def body(i, carry):
            off = base + i * chunk
            pltpu.sync_copy(idx_hbm.at[pl.ds(off, chunk)], idx_v)
            pltpu.async_copy(table_hbm.at[idx_v], rows_v, sem).wait()
            pltpu.sync_copy(rows_v, out_hbm.at[pl.ds(off, chunk)])
            return carry

        lax.fori_loop(0, n_chunks, body, 0)

    return gather_kernel


def kernel(idx, pe):
    B = idx.shape[0] * idx.shape[1]
    D = pe.shape[1]
    idx_flat = idx.reshape(B)
    out = _make_gather(B, D, 512)(idx_flat, pe)
    return out.reshape(idx.shape + (D,))


# speedup vs baseline: 4.2677x; 1.0756x over previous
import functools

import jax
import jax.numpy as jnp
from jax import lax
from jax.experimental import pallas as pl
from jax.experimental.pallas import tpu as pltpu
from jax.experimental.pallas import tpu_sc as plsc

_NC = 2
_NS = 16
_NW = _NC * _NS


def _make_gather(B: int, D: int, chunk: int):
    assert B % (_NW * chunk) == 0
    b_per_w = B // _NW
    n_chunks = b_per_w // chunk
    assert n_chunks >= 3
    mesh = plsc.VectorSubcoreMesh(core_axis_name="c", subcore_axis_name="s")

    @functools.partial(
        pl.kernel,
        mesh=mesh,
        out_type=jax.ShapeDtypeStruct((B, D), jnp.float32),
        scratch_types=[
            pltpu.VMEM((n_chunks, chunk), jnp.int32),
            pltpu.VMEM((3, chunk, D), jnp.float32),
            pltpu.SemaphoreType.DMA((3,)),
            pltpu.SemaphoreType.DMA((3,)),
        ],
        compiler_params=pltpu.CompilerParams(use_tc_tiling_on_sc=False),
    )
    def gather_kernel(idx_hbm, table_hbm, out_hbm, idx_v, rows_v, gsem, ssem):
        wid = lax.axis_index("s") * _NC + lax.axis_index("c")
        base = wid * b_per_w

        pltpu.sync_copy(idx_hbm.at[wid], idx_v)

        def start_gather(i, slot):
            return pltpu.make_async_copy(
                table_hbm.at[idx_v.at[i]], rows_v.at[slot], gsem.at[slot]
            )

        def start_store(i, slot):
            return pltpu.make_async_copy(
                rows_v.at[slot], out_hbm.at[pl.ds(base + i * chunk, chunk)],
                ssem.at[slot],
            )

        start_gather(0, 0).start()
        start_gather(1, 1).start()

        start_gather(0, 0).wait()
        store0 = start_store(0, 0)
        store0.start()
        start_gather(2, 2).start()

        def body(i, carry):
            slot = lax.rem(i, 3)
            nslot = lax.rem(i + 2, 3)
            start_gather(i, slot).wait()
            st = start_store(i, slot)
            st.start()
            start_store(i - 1, nslot).wait()
            start_gather(i + 2, nslot).start()
            return carry

        lax.fori_loop(1, n_chunks - 2, body, 0)

        for i in (n_chunks - 2, n_chunks - 1):
            slot = i % 3
            start_gather(i, slot).wait()
            start_store(i, slot).start()
        for i in (n_chunks - 3, n_chunks - 2, n_chunks - 1):
            start_store(i, i % 3).wait()

    return gather_kernel


def kernel(idx, pe):
    B = idx.shape[0] * idx.shape[1]
    D = pe.shape[1]
    chunk = 512
    idx_w = idx.reshape(_NW, (B // _NW) // chunk, chunk)
    out = _make_gather(B, D, chunk)(idx_w, pe)
    return out.reshape(idx.shape + (D,))


# speedup vs baseline: 4.2780x; 1.0024x over previous
import functools

import jax
import jax.numpy as jnp
from jax import lax
from jax.experimental import pallas as pl
from jax.experimental.pallas import tpu as pltpu
from jax.experimental.pallas import tpu_sc as plsc

_NC = 2
_NS = 16
_NW = _NC * _NS


def _make_gather(S: int, T: int, D: int, K: int):
    assert S % _NW == 0
    n_chunks = S // _NW
    assert n_chunks >= K + 1
    mesh = plsc.VectorSubcoreMesh(core_axis_name="c", subcore_axis_name="s")

    @functools.partial(
        pl.kernel,
        mesh=mesh,
        out_type=jax.ShapeDtypeStruct((S, T, D), jnp.float32),
        scratch_types=[
            pltpu.VMEM((n_chunks, T), jnp.int32),
            pltpu.VMEM((K, T, D), jnp.float32),
            pltpu.SemaphoreType.DMA((K,)),
            pltpu.SemaphoreType.DMA((K,)),
        ],
        compiler_params=pltpu.CompilerParams(use_tc_tiling_on_sc=False),
    )
    def gather_kernel(idx_hbm, table_hbm, out_hbm, idx_v, rows_v, gsem, ssem):
        wid = lax.axis_index("s") * _NC + lax.axis_index("c")
        base = wid * n_chunks

        pltpu.sync_copy(idx_hbm.at[pl.ds(base, n_chunks)], idx_v)

        def gather(i, slot):
            return pltpu.make_async_copy(
                table_hbm.at[idx_v.at[i]],
                rows_v.at[slot], gsem.at[slot],
            )

        def store(i, slot):
            return pltpu.make_async_copy(
                rows_v.at[slot], out_hbm.at[base + i],
                ssem.at[slot],
            )

        for i in range(K - 1):
            gather(i, i).start()

        gather(0, 0).wait()
        store(0, 0).start()
        gather(K - 1, K - 1).start()

        def body(i, carry):
            slot = lax.rem(i, K)
            gather(i, slot).wait()
            store(i, slot).start()
            ns = lax.rem(i + K - 1, K)
            store(i - 1, ns).wait()
            gather(i + K - 1, ns).start()
            return carry

        lax.fori_loop(1, n_chunks - K + 1, body, 0)

        for i in range(n_chunks - K + 1, n_chunks):
            gather(i, i % K).wait()
            store(i, i % K).start()
        for i in range(n_chunks - K, n_chunks):
            store(i, i % K).wait()

    return gather_kernel


def kernel(idx, pe):
    S, T = idx.shape
    D = pe.shape[1]
    return _make_gather(S, T, D, 6)(idx, pe)


# speedup vs baseline: 5.5919x; 1.3071x over previous
import functools

import jax
import jax.numpy as jnp
from jax import lax
from jax.experimental import pallas as pl
from jax.experimental.pallas import tpu as pltpu
from jax.experimental.pallas import tpu_sc as plsc

_NC = 2
_NS = 16
_NW = _NC * _NS


def _make_gather(S: int, T: int, D: int, K: int):
    assert S % _NW == 0
    n_chunks = S // _NW
    assert n_chunks >= K + 1
    mesh = plsc.VectorSubcoreMesh(core_axis_name="c", subcore_axis_name="s")

    @functools.partial(
        pl.kernel,
        mesh=mesh,
        out_type=jax.ShapeDtypeStruct((S, T, 128), jnp.float32),
        scratch_types=[
            pltpu.VMEM((n_chunks * T,), jnp.int32),
            pltpu.VMEM((K, T, 128), jnp.float32),
            pltpu.SemaphoreType.DMA((K,)),
            pltpu.SemaphoreType.DMA((K,)),
        ],
        compiler_params=pltpu.CompilerParams(use_tc_tiling_on_sc=True),
    )
    def gather_kernel(idx_hbm, table_hbm, out_hbm, idx_v, rows_v, gsem, ssem):
        wid = lax.axis_index("s") * _NC + lax.axis_index("c")
        base = wid * n_chunks

        pltpu.sync_copy(idx_hbm.at[pl.ds(base * T, n_chunks * T)], idx_v)

        def gather(i, slot):
            return pltpu.make_async_copy(
                table_hbm.at[idx_v.at[pl.ds(i * T, T)]],
                rows_v.at[slot], gsem.at[slot],
            )

        def store(i, slot):
            return pltpu.make_async_copy(
                rows_v.at[slot], out_hbm.at[base + i],
                ssem.at[slot],
            )

        for i in range(K - 1):
            gather(i, i).start()

        gather(0, 0).wait()
        store(0, 0).start()
        gather(K - 1, K - 1).start()

        def body(i, carry):
            slot = lax.rem(i, K)
            gather(i, slot).wait()
            store(i, slot).start()
            ns = lax.rem(i + K - 1, K)
            store(i - 1, ns).wait()
            gather(i + K - 1, ns).start()
            return carry

        lax.fori_loop(1, n_chunks - K + 1, body, 0)

        for i in range(n_chunks - K + 1, n_chunks):
            gather(i, i % K).wait()
            store(i, i % K).start()
        for i in range(n_chunks - K, n_chunks):
            store(i, i % K).wait()

    return gather_kernel


def kernel(idx, pe):
    S, T = idx.shape
    V, D = pe.shape
    pe_pad = jnp.pad(pe, ((0, 0), (0, 128 - D)))
    idx_flat = idx.reshape(-1)
    out_pad = _make_gather(S, T, D, 3)(idx_flat, pe_pad)
    return out_pad[:, :, :D]


# speedup vs baseline: 5.6052x; 1.0024x over previous
import functools

import jax
import jax.numpy as jnp
from jax import lax
from jax.experimental import pallas as pl
from jax.experimental.pallas import tpu as pltpu
from jax.experimental.pallas import tpu_sc as plsc

_NC = 2
_NS = 16
_NW = _NC * _NS
_L = 128


def _make_gather(R: int, K: int):
    assert R % _NW == 0
    n_chunks = R // _NW
    assert n_chunks >= K + 1
    mesh = plsc.VectorSubcoreMesh(core_axis_name="c", subcore_axis_name="s")

    @functools.partial(
        pl.kernel,
        mesh=mesh,
        out_type=jax.ShapeDtypeStruct((R, _L, _L), jnp.float32),
        scratch_types=[
            pltpu.VMEM((n_chunks, _L), jnp.int32),
            pltpu.VMEM((K, _L, _L), jnp.float32),
            pltpu.SemaphoreType.DMA((K,)),
            pltpu.SemaphoreType.DMA((K,)),
        ],
        compiler_params=pltpu.CompilerParams(use_tc_tiling_on_sc=True),
    )
    def gather_kernel(idx_hbm, table_hbm, out_hbm, idx_v, rows_v, gsem, ssem):
        wid = lax.axis_index("s") * _NC + lax.axis_index("c")
        base = wid * n_chunks

        pltpu.sync_copy(idx_hbm.at[pl.ds(base, n_chunks)], idx_v)

        def gather(i, slot):
            return pltpu.make_async_copy(
                table_hbm.at[idx_v.at[i]],
                rows_v.at[slot], gsem.at[slot],
            )

        def store(i, slot):
            return pltpu.make_async_copy(
                rows_v.at[slot], out_hbm.at[base + i],
                ssem.at[slot],
            )

        for i in range(K - 1):
            gather(i, i).start()

        gather(0, 0).wait()
        store(0, 0).start()
        gather(K - 1, K - 1).start()

        def body(i, carry):
            slot = lax.rem(i, K)
            gather(i, slot).wait()
            store(i, slot).start()
            ns = lax.rem(i + K - 1, K)
            store(i - 1, ns).wait()
            gather(i + K - 1, ns).start()
            return carry

        lax.fori_loop(1, n_chunks - K + 1, body, 0)

        for i in range(n_chunks - K + 1, n_chunks):
            gather(i, i % K).wait()
            store(i, i % K).start()
        for i in range(n_chunks - K, n_chunks):
            store(i, i % K).wait()

    return gather_kernel


def kernel(idx, pe):
    S, T = idx.shape
    V, D = pe.shape
    B = S * T
    pe_pad = jnp.pad(pe, ((0, 0), (0, _L - D)))
    idx_rows = idx.reshape(B // _L, _L)
    out_pad = _make_gather(B // _L, 6)(idx_rows, pe_pad)
    return out_pad.reshape(S, T, _L)[:, :, :D]


# speedup vs baseline: 7.5652x; 1.3497x over previous
import functools

import jax
import jax.numpy as jnp
from jax import lax
from jax.experimental import pallas as pl
from jax.experimental.pallas import tpu as pltpu
from jax.experimental.pallas import tpu_sc as plsc

_NC = 2
_NS = 16
_NW = _NC * _NS
_L = 128


def _make_gather(R: int, D: int, K: int):
    assert R % _NW == 0
    n_chunks = R // _NW
    assert n_chunks >= K + 1
    mesh = plsc.VectorSubcoreMesh(core_axis_name="c", subcore_axis_name="s")

    @functools.partial(
        pl.kernel,
        mesh=mesh,
        out_type=jax.ShapeDtypeStruct((R, _L, _L), jnp.float32),
        scratch_types=[
            pltpu.VMEM((n_chunks, _L), jnp.int32),
            pltpu.VMEM((K, _L, D), jnp.float32),
            pltpu.SemaphoreType.DMA((K,)),
            pltpu.SemaphoreType.DMA((K,)),
        ],
        compiler_params=pltpu.CompilerParams(use_tc_tiling_on_sc=False),
    )
    def gather_kernel(idx_hbm, table_hbm, out_hbm, idx_v, rows_v, gsem, ssem):
        wid = lax.axis_index("s") * _NC + lax.axis_index("c")
        base = wid * n_chunks

        pltpu.sync_copy(idx_hbm.at[pl.ds(base, n_chunks)], idx_v)

        def gather(i, slot):
            return pltpu.make_async_copy(
                table_hbm.at[idx_v.at[i]],
                rows_v.at[slot], gsem.at[slot],
            )

        def store(i, slot):
            return pltpu.make_async_copy(
                rows_v.at[slot],
                out_hbm.at[base + i].at[:, pl.ds(0, D)],
                ssem.at[slot],
            )

        for i in range(K - 1):
            gather(i, i).start()

        gather(0, 0).wait()
        store(0, 0).start()
        gather(K - 1, K - 1).start()

        def body(i, carry):
            slot = lax.rem(i, K)
            gather(i, slot).wait()
            store(i, slot).start()
            ns = lax.rem(i + K - 1, K)
            store(i - 1, ns).wait()
            gather(i + K - 1, ns).start()
            return carry

        lax.fori_loop(1, n_chunks - K + 1, body, 0)

        for i in range(n_chunks - K + 1, n_chunks):
            gather(i, i % K).wait()
            store(i, i % K).start()
        for i in range(n_chunks - K, n_chunks):
            store(i, i % K).wait()

    return gather_kernel


def kernel(idx, pe):
    S, T = idx.shape
    V, D = pe.shape
    B = S * T
    idx_rows = idx.reshape(B // _L, _L)
    out_pad = _make_gather(B // _L, D, 6)(idx_rows, pe)
    return out_pad.reshape(S, T, _L)[:, :, :D]


# speedup vs baseline: 7.5782x; 1.0017x over previous
import functools

import jax
import jax.numpy as jnp
from jax import lax
from jax.experimental import pallas as pl
from jax.experimental.pallas import tpu as pltpu
from jax.experimental.pallas import tpu_sc as plsc

_NC = 2
_NS = 16
_NW = _NC * _NS
_L = 128


def _make_gather(R: int, D: int, K: int):
    assert R % _NW == 0
    n_chunks = R // _NW
    assert n_chunks >= K + 1
    mesh = plsc.VectorSubcoreMesh(core_axis_name="c", subcore_axis_name="s")

    @functools.partial(
        pl.kernel,
        mesh=mesh,
        out_type=jax.ShapeDtypeStruct((R, _L, _L), jnp.float32),
        scratch_types=[
            pltpu.VMEM((n_chunks, _L), jnp.int32),
            pltpu.VMEM((K, _L, D), jnp.float32),
            pltpu.SemaphoreType.DMA((K,)),
            pltpu.SemaphoreType.DMA((K,)),
        ],
        compiler_params=pltpu.CompilerParams(use_tc_tiling_on_sc=False),
    )
    def gather_kernel(idx_hbm, table_hbm, out_hbm, idx_v, rows_v, gsem, ssem):
        wid = lax.axis_index("s") * _NC + lax.axis_index("c")
        base = wid * n_chunks

        pltpu.sync_copy(idx_hbm.at[pl.ds(base, n_chunks)], idx_v)

        def gather(i, slot):
            return pltpu.make_async_copy(
                table_hbm.at[idx_v.at[i]],
                rows_v.at[slot], gsem.at[slot],
            )

        def store(i, slot):
            return pltpu.make_async_copy(
                rows_v.at[slot],
                out_hbm.at[base + i].at[:, pl.ds(0, D)],
                ssem.at[slot],
            )

        for i in range(K - 1):
            gather(i, i).start()

        gather(0, 0).wait()
        store(0, 0).start()
        gather(K - 1, K - 1).start()

        def body(i, carry):
            slot = lax.rem(i, K)
            gather(i, slot).wait()
            store(i, slot).start()
            ns = lax.rem(i + K - 1, K)
            store(i - 1, ns).wait()
            gather(i + K - 1, ns).start()
            return carry

        lax.fori_loop(1, n_chunks - K + 1, body, 0)

        for i in range(n_chunks - K + 1, n_chunks):
            gather(i, i % K).wait()
            store(i, i % K).start()
        for i in range(n_chunks - K, n_chunks):
            store(i, i % K).wait()

    return gather_kernel


def kernel(idx, pe):
    S, T = idx.shape
    V, D = pe.shape
    B = S * T
    idx_rows = idx.reshape(B // _L, _L)
    out_pad = _make_gather(B // _L, D, 10)(idx_rows, pe)
    return out_pad.reshape(S, T, _L)[:, :, :D]
